# baseline (speedup 1.0000x reference)
import functools

import jax
import jax.numpy as jnp
from jax import lax
from jax.experimental import pallas as pl
from jax.experimental.pallas import tpu as pltpu
from jax.experimental.pallas import tpu_sc as plsc

_NC, _NS, _L = 2, 16, 16
_NW = _NC * _NS


def _logits_body(E, x_ref, wc_ref, bc_ref, eps_ref, noisy_ref):
    x = x_ref[...]
    wc = wc_ref[...]
    out = jnp.dot(x, wc, preferred_element_type=jnp.float32) + bc_ref[...]
    out_t = out.T
    eps_t = eps_ref[...].T
    noisy = out_t[:E] + eps_t * jax.nn.softplus(out_t[E:])
    noisy_ref[...] = noisy[None]


def _route_sc(E, CH, K, noisy_hbm, probs_hbm, idx_hbm, stats_hbm,
              nbuf, pbuf, ibuf, sbuf):
    wid = lax.axis_index("s") * _NC + lax.axis_index("c")
    pltpu.sync_copy(noisy_hbm.at[wid], nbuf)

    lane = lax.iota(jnp.int32, _L)
    e_vec = lane & (E - 1)
    pair = lane >> 3
    half = lane >> 1
    parity = lane & 1

    def group(g, acc):
        accp, accc = acc
        base = g * _L
        v = [nbuf[e, pl.ds(base, _L)] for e in range(E)]
        zi = jnp.zeros((_L,), jnp.int32)
        sw = v[1] > v[0]
        m1 = jnp.maximum(v[0], v[1])
        m2 = jnp.minimum(v[0], v[1])
        a1 = jnp.where(sw, zi + 1, zi)
        a2 = jnp.where(sw, zi, zi + 1)
        for e in range(2, E):
            gt1 = v[e] > m1
            gt2 = v[e] > m2
            a2 = jnp.where(gt1, a1, jnp.where(gt2, zi + e, a2))
            m2 = jnp.where(gt1, m1, jnp.where(gt2, v[e], m2))
            a1 = jnp.where(gt1, zi + e, a1)
            m1 = jnp.where(gt1, v[e], m1)
        ex2 = jnp.exp(m2 - m1)
        p1 = 1.0 / (1.0 + ex2)
        p2 = ex2 * p1
        for k in range(_L // E):
            tok = pair + 2 * k
            a1g = a1.at[tok].get(mode="promise_in_bounds")
            a2g = a2.at[tok].get(mode="promise_in_bounds")
            p1g = p1.at[tok].get(mode="promise_in_bounds")
            p2g = p2.at[tok].get(mode="promise_in_bounds")
            s1 = a1g == e_vec
            s2 = a2g == e_vec
            pk = jnp.where(s1, p1g, jnp.where(s2, p2g, jnp.zeros_like(p1g)))
            pbuf[pl.ds(base * E + _L * k, _L)] = pk
            accp = accp + pk
            accc = accc + jnp.where(s1 | s2, 1.0, 0.0).astype(jnp.float32)
        for m in range(_L // E):
            tok = half + E * m
            g1 = a1.at[tok].get(mode="promise_in_bounds")
            g2 = a2.at[tok].get(mode="promise_in_bounds")
            ibuf[pl.ds(base * K + _L * m, _L)] = jnp.where(parity == 0, g1, g2)
        return (accp, accc)

    zf = jnp.zeros((_L,), jnp.float32)
    accp, accc = lax.fori_loop(0, CH // _L, group, (zf, zf))
    sbuf[0, :] = accp
    sbuf[1, :] = accc
    pltpu.sync_copy(pbuf, probs_hbm.at[wid])
    pltpu.sync_copy(ibuf, idx_hbm.at[wid])
    pltpu.sync_copy(sbuf, stats_hbm.at[wid])


def kernel(x, W, b, Wn, bn, noise_eps):
    B, S, DIM = x.shape
    E = W.shape[0]
    K = 2
    N = B * S
    CH = N // _NW
    T = CH

    x2 = x.reshape(N, DIM)
    eps2 = noise_eps.reshape(N, E)
    wc = jnp.concatenate([W, Wn], axis=0).T
    bc = jnp.concatenate([b, bn]).reshape(1, 2 * E)

    noisy = pl.pallas_call(
        functools.partial(_logits_body, E),
        grid=(_NW,),
        in_specs=[
            pl.BlockSpec((T, DIM), lambda i: (i, 0)),
            pl.BlockSpec((DIM, 2 * E), lambda i: (0, 0)),
            pl.BlockSpec((1, 2 * E), lambda i: (0, 0)),
            pl.BlockSpec((T, E), lambda i: (i, 0)),
        ],
        out_specs=pl.BlockSpec((1, E, CH), lambda i: (i, 0, 0)),
        out_shape=jax.ShapeDtypeStruct((_NW, E, CH), jnp.float32),
        compiler_params=pltpu.CompilerParams(
            dimension_semantics=("arbitrary",),
        ),
    )(x2, wc, bc, eps2)

    mesh = plsc.VectorSubcoreMesh(core_axis_name="c", subcore_axis_name="s")
    route = pl.kernel(
        functools.partial(_route_sc, E, CH, K),
        out_type=[
            jax.ShapeDtypeStruct((_NW, CH * E), jnp.float32),
            jax.ShapeDtypeStruct((_NW, CH * K), jnp.int32),
            jax.ShapeDtypeStruct((_NW, 2, _L), jnp.float32),
        ],
        mesh=mesh,
        scratch_types=[
            pltpu.VMEM((E, CH), jnp.float32),
            pltpu.VMEM((CH * E,), jnp.float32),
            pltpu.VMEM((CH * K,), jnp.int32),
            pltpu.VMEM((2, _L), jnp.float32),
        ],
    )
    probs_w, idx_w, stats = route(noisy)

    probs = probs_w.reshape(B, S, E)
    idx = idx_w.reshape(B, S, K)
    psum = stats[:, 0, :].sum(axis=0)
    csum = stats[:, 1, :].sum(axis=0)
    prob_mean = (psum[:E] + psum[E:]) / N
    prob_count = (csum[:E] + csum[E:]) / N
    lb_loss = E * jnp.sum(prob_mean * prob_count)
    return (probs, idx, lb_loss)


# speedup vs baseline: 1.3408x; 1.3408x over previous
import functools

import jax
import jax.numpy as jnp
from jax import lax
from jax.experimental import pallas as pl
from jax.experimental.pallas import tpu as pltpu
from jax.experimental.pallas import tpu_sc as plsc

_NC, _NS, _L = 2, 16, 16
_NW = _NC * _NS


def _logits_body(E, x_ref, wc_ref, bc_ref, eps_ref, noisy_ref):
    x = x_ref[...]
    wc = wc_ref[...]
    out = jnp.dot(x, wc, preferred_element_type=jnp.float32) + bc_ref[...]
    out_t = out.T
    eps_t = eps_ref[...].T
    noisy = out_t[:E] + eps_t * jax.nn.softplus(out_t[E:])
    noisy_ref[...] = noisy[None]


def _route_sc(E, CH, noisy_hbm, probs_hbm, idx_hbm, stats_hbm,
              nbuf, pbuf, ibuf, sbuf):
    wid = lax.axis_index("s") * _NC + lax.axis_index("c")
    pltpu.sync_copy(noisy_hbm.at[wid], nbuf)

    def group(g, acc):
        accp, accc = acc
        base = g * _L
        v = [nbuf[e, pl.ds(base, _L)] for e in range(E)]
        zi = jnp.zeros((_L,), jnp.int32)
        zf = jnp.zeros((_L,), jnp.float32)
        sw = v[1] > v[0]
        m1 = jnp.maximum(v[0], v[1])
        m2 = jnp.minimum(v[0], v[1])
        a1 = jnp.where(sw, zi + 1, zi)
        a2 = jnp.where(sw, zi, zi + 1)
        for e in range(2, E):
            gt1 = v[e] > m1
            gt2 = v[e] > m2
            a2 = jnp.where(gt1, a1, jnp.where(gt2, zi + e, a2))
            m2 = jnp.where(gt1, m1, jnp.where(gt2, v[e], m2))
            a1 = jnp.where(gt1, zi + e, a1)
            m1 = jnp.where(gt1, v[e], m1)
        ex2 = jnp.exp(m2 - m1)
        p1 = 1.0 / (1.0 + ex2)
        p2 = ex2 * p1
        newp, newc = [], []
        for e in range(E):
            s1 = a1 == e
            s2 = a2 == e
            pe = jnp.where(s1, p1, jnp.where(s2, p2, zf))
            pbuf[e, pl.ds(base, _L)] = pe
            newp.append(accp[e] + pe)
            newc.append(accc[e] + jnp.where(s1 | s2, zf + 1.0, zf))
        ibuf[0, pl.ds(base, _L)] = a1
        ibuf[1, pl.ds(base, _L)] = a2
        return (newp, newc)

    zf = jnp.zeros((_L,), jnp.float32)
    accp, accc = lax.fori_loop(0, CH // _L, group, ([zf] * E, [zf] * E))
    for e in range(E):
        sbuf[e, :] = accp[e]
        sbuf[E + e, :] = accc[e]
    pltpu.sync_copy(pbuf, probs_hbm.at[wid])
    pltpu.sync_copy(ibuf, idx_hbm.at[wid])
    pltpu.sync_copy(sbuf, stats_hbm.at[wid])


def kernel(x, W, b, Wn, bn, noise_eps):
    B, S, DIM = x.shape
    E = W.shape[0]
    K = 2
    N = B * S
    CH = N // _NW
    T = CH

    x2 = x.reshape(N, DIM)
    eps2 = noise_eps.reshape(N, E)
    wc = jnp.concatenate([W, Wn], axis=0).T
    bc = jnp.concatenate([b, bn]).reshape(1, 2 * E)

    noisy = pl.pallas_call(
        functools.partial(_logits_body, E),
        grid=(_NW,),
        in_specs=[
            pl.BlockSpec((T, DIM), lambda i: (i, 0)),
            pl.BlockSpec((DIM, 2 * E), lambda i: (0, 0)),
            pl.BlockSpec((1, 2 * E), lambda i: (0, 0)),
            pl.BlockSpec((T, E), lambda i: (i, 0)),
        ],
        out_specs=pl.BlockSpec((1, E, CH), lambda i: (i, 0, 0)),
        out_shape=jax.ShapeDtypeStruct((_NW, E, CH), jnp.float32),
        compiler_params=pltpu.CompilerParams(
            dimension_semantics=("arbitrary",),
        ),
    )(x2, wc, bc, eps2)

    mesh = plsc.VectorSubcoreMesh(core_axis_name="c", subcore_axis_name="s")
    route = pl.kernel(
        functools.partial(_route_sc, E, CH),
        out_type=[
            jax.ShapeDtypeStruct((_NW, E, CH), jnp.float32),
            jax.ShapeDtypeStruct((_NW, K, CH), jnp.int32),
            jax.ShapeDtypeStruct((_NW, 2 * E, _L), jnp.float32),
        ],
        mesh=mesh,
        scratch_types=[
            pltpu.VMEM((E, CH), jnp.float32),
            pltpu.VMEM((E, CH), jnp.float32),
            pltpu.VMEM((K, CH), jnp.int32),
            pltpu.VMEM((2 * E, _L), jnp.float32),
        ],
    )
    probs_w, idx_w, stats = route(noisy)

    probs = probs_w.transpose(0, 2, 1).reshape(B, S, E)
    idx = idx_w.transpose(0, 2, 1).reshape(B, S, K)
    prob_mean = stats[:, :E, :].sum(axis=(0, 2)) / N
    prob_count = stats[:, E:, :].sum(axis=(0, 2)) / N
    lb_loss = E * jnp.sum(prob_mean * prob_count)
    return (probs, idx, lb_loss)
